# baseline (speedup 1.0000x reference)
import functools

import jax
import jax.numpy as jnp
from jax import lax
from jax.experimental import pallas as pl
from jax.experimental.pallas import tpu as pltpu
from jax.experimental.pallas import tpu_sc as plsc

NW = 32
CHUNK = 32
LANES = 16


def _emb_body(seq_len, x_hbm, tok_hbm, pos_hbm, out_hbm, idx_v, tok_v, pos_v, sem):
    n_chunks = idx_v.shape[0]
    d = tok_v.shape[1]
    wid = lax.axis_index("s") * 2 + lax.axis_index("c")
    t_per_w = n_chunks * CHUNK
    base = wid * t_per_w
    pos0 = lax.bitwise_and(base, seq_len - 1)
    pltpu.sync_copy(x_hbm.at[wid], idx_v)

    def chunk_body(ci, carry):
        off = ci * CHUNK
        gather = pltpu.async_copy(tok_hbm.at[idx_v.at[ci]], tok_v, sem)
        pltpu.sync_copy(pos_hbm.at[pl.ds(pl.multiple_of(pos0 + off, 8), CHUNK)], pos_v)
        gather.wait()

        def row_body(r, c2):
            for j in range(d // LANES):
                v = pos_v[r, pl.ds(j * LANES, LANES)]
                plsc.addupdate(tok_v.at[r, pl.ds(j * LANES, LANES)], v)
            return c2

        lax.fori_loop(0, CHUNK, row_body, 0)
        pltpu.sync_copy(tok_v, out_hbm.at[pl.ds(base + off, CHUNK)])
        return carry

    lax.fori_loop(0, n_chunks, chunk_body, 0)


def kernel(x, tok_emb, pos_emb):
    b, s = x.shape
    d = tok_emb.shape[1]
    t = b * s
    t_per_w = t // NW
    n_chunks = t_per_w // CHUNK
    xf = x.reshape(NW, n_chunks, CHUNK).astype(jnp.int32)
    assert s & (s - 1) == 0 and s % t_per_w == 0

    mesh = plsc.VectorSubcoreMesh(core_axis_name="c", subcore_axis_name="s")
    emb = functools.partial(
        pl.kernel,
        out_type=jax.ShapeDtypeStruct((t, d), jnp.float32),
        mesh=mesh,
        scratch_types=[
            pltpu.VMEM((n_chunks, CHUNK), jnp.int32),
            pltpu.VMEM((CHUNK, d), jnp.float32),
            pltpu.VMEM((CHUNK, d), jnp.float32),
            pltpu.SemaphoreType.DMA,
        ],
    )(functools.partial(_emb_body, s))
    out = emb(xf, tok_emb, pos_emb)
    return out.reshape(b, s, d)


# speedup vs baseline: 1.1608x; 1.1608x over previous
import functools

import jax
import jax.numpy as jnp
from jax import lax
from jax.experimental import pallas as pl
from jax.experimental.pallas import tpu as pltpu
from jax.experimental.pallas import tpu_sc as plsc

NW = 32
CHUNK = 32
LANES = 16


def _emb_body(b, s, x_hbm, tok_hbm, pos_hbm, out_hbm,
              idx_v, tok0, tok1, pos_v, gs0, gs1, ws0, ws1):
    d = tok0.shape[1]
    n_chunks = idx_v.shape[0]
    n_halves = n_chunks // b
    wid = lax.axis_index("s") * 2 + lax.axis_index("c")
    pos_base = wid * (n_halves * CHUNK)
    pltpu.sync_copy(x_hbm.at[wid], idx_v)

    tok = [tok0, tok1]
    gsem = [gs0, gs1]
    wsem = [ws0, ws1]
    g_desc = [None, None]
    w_desc = [None, None]

    def out_slice(ci):
        half, bi = divmod(ci, b)
        off = pos_base + (bi * s + half * CHUNK)
        return out_hbm.at[pl.ds(pl.multiple_of(off, 8), CHUNK)]

    for ci in range(n_chunks):
        sl = ci % 2
        if ci == 0:
            g_desc[0] = pltpu.async_copy(tok_hbm.at[idx_v.at[0]], tok[0], gsem[0])
        if ci % b == 0:
            half = ci // b
            off = pos_base + half * CHUNK
            pltpu.sync_copy(
                pos_hbm.at[pl.ds(pl.multiple_of(off, 8), CHUNK)], pos_v)
        if ci + 1 < n_chunks:
            s2 = (ci + 1) % 2
            if w_desc[s2] is not None:
                w_desc[s2].wait()
            g_desc[s2] = pltpu.async_copy(
                tok_hbm.at[idx_v.at[ci + 1]], tok[s2], gsem[s2])
        g_desc[sl].wait()
        cur = tok[sl]

        def row_body(r, c2, cur=cur):
            for j in range(d // LANES):
                v = pos_v[r, pl.ds(j * LANES, LANES)]
                plsc.addupdate(cur.at[r, pl.ds(j * LANES, LANES)], v)
            return c2

        lax.fori_loop(0, CHUNK, row_body, 0)
        w_desc[sl] = pltpu.async_copy(cur, out_slice(ci), wsem[sl])

    for sl in range(2):
        if w_desc[sl] is not None:
            w_desc[sl].wait()


def kernel(x, tok_emb, pos_emb):
    b, s = x.shape
    d = tok_emb.shape[1]
    t = b * s
    t_per_w = t // NW
    pos_per_w = s // NW
    n_halves = pos_per_w // CHUNK
    n_chunks = n_halves * b
    assert n_halves * CHUNK * b == t_per_w and d % LANES == 0
    xf = (x.reshape(b, NW, n_halves, CHUNK)
           .transpose(1, 2, 0, 3)
           .reshape(NW, n_chunks, CHUNK)
           .astype(jnp.int32))

    mesh = plsc.VectorSubcoreMesh(core_axis_name="c", subcore_axis_name="s")
    emb = functools.partial(
        pl.kernel,
        out_type=jax.ShapeDtypeStruct((t, d), jnp.float32),
        mesh=mesh,
        scratch_types=[
            pltpu.VMEM((n_chunks, CHUNK), jnp.int32),
            pltpu.VMEM((CHUNK, d), jnp.float32),
            pltpu.VMEM((CHUNK, d), jnp.float32),
            pltpu.VMEM((CHUNK, d), jnp.float32),
            pltpu.SemaphoreType.DMA,
            pltpu.SemaphoreType.DMA,
            pltpu.SemaphoreType.DMA,
            pltpu.SemaphoreType.DMA,
        ],
    )(functools.partial(_emb_body, b, s))
    out = emb(xf, tok_emb, pos_emb)
    return out.reshape(b, s, d)


# speedup vs baseline: 2.1657x; 1.8657x over previous
import functools

import jax
import jax.numpy as jnp
from jax import lax
from jax.experimental import pallas as pl
from jax.experimental.pallas import tpu as pltpu
from jax.experimental.pallas import tpu_sc as plsc

NW = 32
CHUNK = 32
LANES = 16


def _emb_body(b, s, x_hbm, tok_hbm, pos_hbm, out_hbm,
              idx_v, tok0, tok1, pos_v, gs0, gs1, ws0, ws1):
    d = tok0.shape[1]
    n_chunks = idx_v.shape[0]
    n_halves = n_chunks // b
    wid = lax.axis_index("s") * 2 + lax.axis_index("c")
    pos_base = wid * (n_halves * CHUNK)
    pltpu.sync_copy(x_hbm.at[wid], idx_v)

    tok = [tok0, tok1]
    gsem = [gs0, gs1]
    wsem = [ws0, ws1]
    g_desc = [None, None]
    w_desc = [None, None]

    def out_slice(ci):
        half, bi = divmod(ci, b)
        off = pos_base + (bi * s + half * CHUNK)
        return out_hbm.at[pl.ds(pl.multiple_of(off, 8), CHUNK)]

    for ci in range(n_chunks):
        sl = ci % 2
        if ci == 0:
            g_desc[0] = pltpu.async_copy(tok_hbm.at[idx_v.at[0]], tok[0], gsem[0])
        if ci % b == 0:
            half = ci // b
            off = pos_base + half * CHUNK
            pltpu.sync_copy(
                pos_hbm.at[pl.ds(pl.multiple_of(off, 8), CHUNK)], pos_v)
        if ci + 1 < n_chunks:
            s2 = (ci + 1) % 2
            if w_desc[s2] is not None:
                w_desc[s2].wait()
            g_desc[s2] = pltpu.async_copy(
                tok_hbm.at[idx_v.at[ci + 1]], tok[s2], gsem[s2])
        g_desc[sl].wait()
        cur = tok[sl]
        gpr = d // LANES

        @plsc.parallel_loop(0, CHUNK * gpr, unroll=8)
        def add_body(g, cur=cur):
            r = lax.shift_right_logical(g, gpr.bit_length() - 1)
            col = pl.multiple_of(
                lax.shift_left(lax.bitwise_and(g, gpr - 1), 4), LANES)
            v = pos_v[r, pl.ds(col, LANES)]
            plsc.addupdate(cur.at[r, pl.ds(col, LANES)], v)

        w_desc[sl] = pltpu.async_copy(cur, out_slice(ci), wsem[sl])

    for sl in range(2):
        if w_desc[sl] is not None:
            w_desc[sl].wait()


def kernel(x, tok_emb, pos_emb):
    b, s = x.shape
    d = tok_emb.shape[1]
    t = b * s
    t_per_w = t // NW
    pos_per_w = s // NW
    n_halves = pos_per_w // CHUNK
    n_chunks = n_halves * b
    assert n_halves * CHUNK * b == t_per_w and d % LANES == 0
    xf = (x.reshape(b, NW, n_halves, CHUNK)
           .transpose(1, 2, 0, 3)
           .reshape(NW, n_chunks, CHUNK)
           .astype(jnp.int32))

    mesh = plsc.VectorSubcoreMesh(core_axis_name="c", subcore_axis_name="s")
    emb = functools.partial(
        pl.kernel,
        out_type=jax.ShapeDtypeStruct((t, d), jnp.float32),
        mesh=mesh,
        scratch_types=[
            pltpu.VMEM((n_chunks, CHUNK), jnp.int32),
            pltpu.VMEM((CHUNK, d), jnp.float32),
            pltpu.VMEM((CHUNK, d), jnp.float32),
            pltpu.VMEM((CHUNK, d), jnp.float32),
            pltpu.SemaphoreType.DMA,
            pltpu.SemaphoreType.DMA,
            pltpu.SemaphoreType.DMA,
            pltpu.SemaphoreType.DMA,
        ],
    )(functools.partial(_emb_body, b, s))
    out = emb(xf, tok_emb, pos_emb)
    return out.reshape(b, s, d)
